# baseline (speedup 1.0000x reference)
import jax
import jax.numpy as jnp
from jax.experimental import pallas as pl
from jax.experimental.pallas import tpu as pltpu

DIM = 4096
NUM_EXPERTS = 64
TOP_K = 8
TOKENS = 16384

BT = 1024
NBLK = TOKENS // BT


def _router_body(x_ref, w_ref, b_ref, n_ref, out_ref, idx_ref, nscr):
    i = pl.program_id(0)
    wr = jax.lax.rem(i, 2)
    rd = 1 - wr

    acc = jnp.dot(x_ref[...], w_ref[...], preferred_element_type=jnp.float32)
    acc = acc + b_ref[...]
    logits = acc[:, :NUM_EXPERTS]
    nl = acc[:, NUM_EXPERTS:]
    sp = jnp.maximum(nl, 0.0) + jnp.log1p(jnp.exp(-jnp.abs(nl)))
    nscr[pl.ds(wr, 1), :, :] = (logits + n_ref[...] * sp)[None]

    noisy = nscr[pl.ds(rd, 1), :, :][0]
    iotaf = jax.lax.broadcasted_iota(
        jnp.int32, (BT, NUM_EXPERTS), 1).astype(jnp.float32)
    work = noisy
    v0 = None
    idxs = []
    for k in range(TOP_K):
        m = jnp.max(work, axis=1, keepdims=True)
        if k == 0:
            v0 = m
        t = jnp.where(work == m, iotaf, jnp.float32(NUM_EXPERTS))
        idxf = jnp.min(t, axis=1, keepdims=True)
        idxs.append(idxf)
        work = jnp.where(t == idxf, -jnp.inf, work)

    sel = work == -jnp.inf
    p = jnp.where(sel, jnp.exp(noisy - v0), 0.0)
    denom = jnp.sum(p, axis=1, keepdims=True)
    out_ref[...] = p / denom
    idx_ref[...] = jnp.concatenate(idxs, axis=1).astype(jnp.int32)


@jax.jit
def kernel(x, W_route, b_route, W_noise, b_noise, noise):
    w = jnp.concatenate([W_route, W_noise], axis=0).T
    b = jnp.concatenate([b_route, b_noise])[None, :]
    last = NBLK - 1
    out, idx = pl.pallas_call(
        _router_body,
        grid=(NBLK + 1,),
        in_specs=[
            pl.BlockSpec((BT, DIM), lambda i: (jnp.minimum(i, last), 0)),
            pl.BlockSpec((DIM, 2 * NUM_EXPERTS), lambda i: (0, 0)),
            pl.BlockSpec((1, 2 * NUM_EXPERTS), lambda i: (0, 0)),
            pl.BlockSpec((BT, NUM_EXPERTS), lambda i: (jnp.minimum(i, last), 0)),
        ],
        out_specs=[
            pl.BlockSpec((BT, NUM_EXPERTS), lambda i: (jnp.maximum(i - 1, 0), 0)),
            pl.BlockSpec((BT, TOP_K), lambda i: (jnp.maximum(i - 1, 0), 0)),
        ],
        out_shape=[
            jax.ShapeDtypeStruct((TOKENS, NUM_EXPERTS), jnp.float32),
            jax.ShapeDtypeStruct((TOKENS, TOP_K), jnp.int32),
        ],
        scratch_shapes=[pltpu.VMEM((2, BT, NUM_EXPERTS), jnp.float32)],
        compiler_params=pltpu.CompilerParams(
            dimension_semantics=("arbitrary",),
        ),
    )(x, w, b, noise)
    return (out, idx)


# speedup vs baseline: 1.2475x; 1.2475x over previous
import jax
import jax.numpy as jnp
from jax.experimental import pallas as pl
from jax.experimental.pallas import tpu as pltpu

DIM = 4096
NUM_EXPERTS = 64
TOP_K = 8
TOKENS = 16384

BT = 1024


def _router_body(x_ref, w_ref, b_ref, nT_ref, eye_ref, out_ref, idx_ref):
    accT = jax.lax.dot_general(
        w_ref[...], x_ref[...], (((1,), (1,)), ((), ())),
        preferred_element_type=jnp.float32)
    accT = accT + b_ref[...]
    logits = accT[:NUM_EXPERTS, :]
    nl = accT[NUM_EXPERTS:, :]
    sp = jnp.maximum(nl, 0.0) + jnp.log1p(jnp.exp(-jnp.abs(nl)))
    noisy = logits + nT_ref[...] * sp

    iotaf = jax.lax.broadcasted_iota(
        jnp.int32, (NUM_EXPERTS, BT), 0).astype(jnp.float32)
    work = noisy
    v0 = None
    idxs = []
    for k in range(TOP_K):
        m = jnp.max(work, axis=0, keepdims=True)
        if k == 0:
            v0 = m
        t = jnp.where(work == m, iotaf, jnp.float32(NUM_EXPERTS))
        idxf = jnp.min(t, axis=0, keepdims=True)
        idxs.append(idxf)
        work = jnp.where(t == idxf, -jnp.inf, work)

    sel = work == -jnp.inf
    p = jnp.where(sel, jnp.exp(noisy - v0), 0.0)
    denom = jnp.sum(p, axis=0, keepdims=True)
    pn = p / denom

    eye = eye_ref[...]
    out_ref[...] = jax.lax.dot_general(
        pn, eye, (((0,), (0,)), ((), ())),
        preferred_element_type=jnp.float32)
    idxT = jnp.concatenate(idxs, axis=0)
    idx_ref[...] = jax.lax.dot_general(
        idxT, eye[:TOP_K, :TOP_K], (((0,), (0,)), ((), ())),
        preferred_element_type=jnp.float32).astype(jnp.int32)


@jax.jit
def kernel(x, W_route, b_route, W_noise, b_noise, noise):
    w = jnp.concatenate([W_route, W_noise], axis=0)
    b = jnp.concatenate([b_route, b_noise])[:, None]
    nT = noise.T
    eye = jnp.eye(NUM_EXPERTS, dtype=jnp.float32)
    grid = (TOKENS // BT,)
    out, idx = pl.pallas_call(
        _router_body,
        grid=grid,
        in_specs=[
            pl.BlockSpec((BT, DIM), lambda i: (i, 0)),
            pl.BlockSpec((2 * NUM_EXPERTS, DIM), lambda i: (0, 0)),
            pl.BlockSpec((2 * NUM_EXPERTS, 1), lambda i: (0, 0)),
            pl.BlockSpec((NUM_EXPERTS, BT), lambda i: (0, i)),
            pl.BlockSpec((NUM_EXPERTS, NUM_EXPERTS), lambda i: (0, 0)),
        ],
        out_specs=[
            pl.BlockSpec((BT, NUM_EXPERTS), lambda i: (i, 0)),
            pl.BlockSpec((BT, TOP_K), lambda i: (i, 0)),
        ],
        out_shape=[
            jax.ShapeDtypeStruct((TOKENS, NUM_EXPERTS), jnp.float32),
            jax.ShapeDtypeStruct((TOKENS, TOP_K), jnp.int32),
        ],
        compiler_params=pltpu.CompilerParams(
            dimension_semantics=("arbitrary",),
        ),
    )(x, w, b, nT, eye)
    return (out, idx)
